# baseline (speedup 1.0000x reference)
import jax
import jax.numpy as jnp
from jax.experimental import pallas as pl
from jax.experimental.pallas import tpu as pltpu

_D = 1024
_M = 4096
_K = 5
_LBLK = 256


def _lmm_block_kernel(enc_ref, mem_ref, out_ref, memn_ref):
    mem = mem_ref[...]

    @pl.when(pl.program_id(0) == 0)
    def _normalize_memory():
        ssq = jnp.sum(mem * mem, axis=1, keepdims=True)
        n = jnp.maximum(jnp.sqrt(ssq), 1e-12)
        memn_ref[...] = (mem / n).astype(jnp.bfloat16)

    half = _LBLK // 2
    for h in range(2):
        enc = enc_ref[pl.ds(h * half, half), :]
        essq = jnp.sum(enc * enc, axis=1, keepdims=True)
        en = (enc / jnp.maximum(jnp.sqrt(essq), 1e-12)).astype(jnp.bfloat16)

        sim = jax.lax.dot_general(
            en, memn_ref[...], (((1,), (1,)), ((), ())),
            preferred_element_type=jnp.float32)

        work = sim
        neg = jnp.float32(-jnp.inf)
        mx = jnp.max(work, axis=1, keepdims=True)
        for _ in range(_K - 1):
            work = jnp.where(work == mx, neg, work)
            mx = jnp.max(work, axis=1, keepdims=True)

        mask = (sim >= mx).astype(jnp.float32)
        cnt = jnp.sum(mask, axis=1, keepdims=True)
        matched = jax.lax.dot_general(
            mask, mem, (((1,), (0,)), ((), ())),
            preferred_element_type=jnp.float32)
        out_ref[pl.ds(h * half, half), :] = enc + matched / cnt


def kernel(encoded, memory):
    B, L, D = encoded.shape
    M = memory.shape[0]
    x2d = encoded.reshape(B * L, D)
    n_blocks = (B * L) // _LBLK

    out = pl.pallas_call(
        _lmm_block_kernel,
        grid=(n_blocks,),
        in_specs=[
            pl.BlockSpec((_LBLK, D), lambda i: (i, 0)),
            pl.BlockSpec((M, D), lambda i: (0, 0)),
        ],
        out_specs=pl.BlockSpec((_LBLK, D), lambda i: (i, 0)),
        out_shape=jax.ShapeDtypeStruct((B * L, D), jnp.float32),
        scratch_shapes=[pltpu.VMEM((M, D), jnp.bfloat16)],
        compiler_params=pltpu.CompilerParams(
            vmem_limit_bytes=100 * 1024 * 1024),
    )(x2d, memory)
    return out.reshape(B, L, D)


# speedup vs baseline: 1.6373x; 1.6373x over previous
import jax
import jax.numpy as jnp
from jax.experimental import pallas as pl
from jax.experimental.pallas import tpu as pltpu

_D = 1024
_M = 4096
_K = 5
_LBLK = 256


def _lmm_block_kernel(enc_ref, mem_ref, out_ref, memn_ref, memb_ref):
    @pl.when(pl.program_id(0) == 0)
    def _normalize_memory():
        mem = mem_ref[...]
        ssq = jnp.sum(mem * mem, axis=1, keepdims=True)
        n = jnp.maximum(jnp.sqrt(ssq), 1e-12)
        memn_ref[...] = (mem / n).astype(jnp.bfloat16)
        memb_ref[...] = mem.astype(jnp.bfloat16)

    enc = enc_ref[...]
    essq = jnp.sum(enc * enc, axis=1, keepdims=True)
    en = (enc / jnp.maximum(jnp.sqrt(essq), 1e-12)).astype(jnp.bfloat16)

    sim = jax.lax.dot_general(
        en, memn_ref[...], (((1,), (1,)), ((), ())),
        preferred_element_type=jnp.float32)

    work = sim
    neg = jnp.float32(-jnp.inf)
    mx = jnp.max(work, axis=1, keepdims=True)
    for _ in range(_K - 1):
        work = jnp.where(work == mx, neg, work)
        mx = jnp.max(work, axis=1, keepdims=True)

    maskb = (sim >= mx).astype(jnp.bfloat16)
    cnt = jnp.sum(maskb.astype(jnp.float32), axis=1, keepdims=True)
    matched = jax.lax.dot_general(
        maskb, memb_ref[...], (((1,), (0,)), ((), ())),
        preferred_element_type=jnp.float32)
    out_ref[...] = enc + matched / cnt


def kernel(encoded, memory):
    B, L, D = encoded.shape
    M = memory.shape[0]
    x2d = encoded.reshape(B * L, D)
    n_blocks = (B * L) // _LBLK

    out = pl.pallas_call(
        _lmm_block_kernel,
        grid=(n_blocks,),
        in_specs=[
            pl.BlockSpec((_LBLK, D), lambda i: (i, 0)),
            pl.BlockSpec((M, D), lambda i: (0, 0)),
        ],
        out_specs=pl.BlockSpec((_LBLK, D), lambda i: (i, 0)),
        out_shape=jax.ShapeDtypeStruct((B * L, D), jnp.float32),
        scratch_shapes=[pltpu.VMEM((M, D), jnp.bfloat16),
                        pltpu.VMEM((M, D), jnp.bfloat16)],
        compiler_params=pltpu.CompilerParams(
            vmem_limit_bytes=100 * 1024 * 1024),
    )(x2d, memory)
    return out.reshape(B, L, D)


# speedup vs baseline: 1.7019x; 1.0395x over previous
import jax
import jax.numpy as jnp
from jax.experimental import pallas as pl
from jax.experimental.pallas import tpu as pltpu

_D = 1024
_M = 4096
_K = 5
_LBLK = 512


def _lmm_block_kernel(enc_ref, mem_ref, out_ref, memn_ref, memb_ref):
    @pl.when(pl.program_id(0) == 0)
    def _normalize_memory():
        mem = mem_ref[...]
        ssq = jnp.sum(mem * mem, axis=1, keepdims=True)
        n = jnp.maximum(jnp.sqrt(ssq), 1e-12)
        memn_ref[...] = (mem / n).astype(jnp.bfloat16)
        memb_ref[...] = mem.astype(jnp.bfloat16)

    enc = enc_ref[...]
    essq = jnp.sum(enc * enc, axis=1, keepdims=True)
    en = (enc / jnp.maximum(jnp.sqrt(essq), 1e-12)).astype(jnp.bfloat16)

    sim = jax.lax.dot_general(
        en, memn_ref[...], (((1,), (1,)), ((), ())),
        preferred_element_type=jnp.float32)

    work = sim
    neg = jnp.float32(-jnp.inf)
    mx = jnp.max(work, axis=1, keepdims=True)
    for _ in range(_K - 1):
        work = jnp.where(work == mx, neg, work)
        mx = jnp.max(work, axis=1, keepdims=True)

    maskb = (sim >= mx).astype(jnp.bfloat16)
    cnt = jnp.sum(maskb.astype(jnp.float32), axis=1, keepdims=True)
    matched = jax.lax.dot_general(
        maskb, memb_ref[...], (((1,), (0,)), ((), ())),
        preferred_element_type=jnp.float32)
    out_ref[...] = enc + matched / cnt


def kernel(encoded, memory):
    B, L, D = encoded.shape
    M = memory.shape[0]
    x2d = encoded.reshape(B * L, D)
    n_blocks = (B * L) // _LBLK

    out = pl.pallas_call(
        _lmm_block_kernel,
        grid=(n_blocks,),
        in_specs=[
            pl.BlockSpec((_LBLK, D), lambda i: (i, 0)),
            pl.BlockSpec((M, D), lambda i: (0, 0)),
        ],
        out_specs=pl.BlockSpec((_LBLK, D), lambda i: (i, 0)),
        out_shape=jax.ShapeDtypeStruct((B * L, D), jnp.float32),
        scratch_shapes=[pltpu.VMEM((M, D), jnp.bfloat16),
                        pltpu.VMEM((M, D), jnp.bfloat16)],
        compiler_params=pltpu.CompilerParams(
            vmem_limit_bytes=100 * 1024 * 1024),
    )(x2d, memory)
    return out.reshape(B, L, D)


# speedup vs baseline: 1.7034x; 1.0008x over previous
import jax
import jax.numpy as jnp
from jax.experimental import pallas as pl
from jax.experimental.pallas import tpu as pltpu

_D = 1024
_M = 4096
_K = 5
_LBLK = 512


def _lmm_block_kernel(enc_ref, mem_ref, out_ref, memn_ref, memb_ref):
    @pl.when(pl.program_id(0) == 0)
    def _normalize_memory():
        mem = mem_ref[...]
        ssq = jnp.sum(mem * mem, axis=1, keepdims=True)
        n = jnp.maximum(jnp.sqrt(ssq), 1e-12)
        memn_ref[...] = (mem / n).astype(jnp.bfloat16)
        memb_ref[...] = mem.astype(jnp.bfloat16)

    enc = enc_ref[...]
    essq = jnp.sum(enc * enc, axis=1, keepdims=True)
    en = (enc / jnp.maximum(jnp.sqrt(essq), 1e-12)).astype(jnp.bfloat16)

    sim = jax.lax.dot_general(
        en, memn_ref[...], (((1,), (1,)), ((), ())),
        preferred_element_type=jnp.float32)

    neg = jnp.float32(-jnp.inf)
    mx = jnp.max(sim, axis=1, keepdims=True)
    for _ in range(_K - 1):
        mx = jnp.max(jnp.where(sim < mx, sim, neg), axis=1, keepdims=True)

    maskb = (sim >= mx).astype(jnp.bfloat16)
    cnt = jnp.sum(maskb.astype(jnp.float32), axis=1, keepdims=True)
    matched = jax.lax.dot_general(
        maskb, memb_ref[...], (((1,), (0,)), ((), ())),
        preferred_element_type=jnp.float32)
    out_ref[...] = enc + matched / cnt


def kernel(encoded, memory):
    B, L, D = encoded.shape
    M = memory.shape[0]
    x2d = encoded.reshape(B * L, D)
    n_blocks = (B * L) // _LBLK

    out = pl.pallas_call(
        _lmm_block_kernel,
        grid=(n_blocks,),
        in_specs=[
            pl.BlockSpec((_LBLK, D), lambda i: (i, 0)),
            pl.BlockSpec((M, D), lambda i: (0, 0)),
        ],
        out_specs=pl.BlockSpec((_LBLK, D), lambda i: (i, 0)),
        out_shape=jax.ShapeDtypeStruct((B * L, D), jnp.float32),
        scratch_shapes=[pltpu.VMEM((M, D), jnp.bfloat16),
                        pltpu.VMEM((M, D), jnp.bfloat16)],
        compiler_params=pltpu.CompilerParams(
            vmem_limit_bytes=124 * 1024 * 1024),
    )(x2d, memory)
    return out.reshape(B, L, D)
